# Initial kernel scaffold; baseline (speedup 1.0000x reference)
import jax
import jax.numpy as jnp
from jax.experimental import pallas as pl


def kernel(x, edge_index, W1l, b1, W1r, g1, be1, W2l, b2, W2r, g2, be2, W3l, b3, W3r):
    raise NotImplementedError("write your pallas kernel here")


# speedup vs baseline: 5.4804x; 5.4804x over previous
import functools

import jax
import jax.numpy as jnp
import numpy as np
from jax import lax
from jax.experimental import pallas as pl
from jax.experimental.pallas import tpu as pltpu
from jax.experimental.pallas import tpu_sc as plsc

_N = 10000
_E = 320000
_D = 128
_EPS = 1e-5

_NC = 2
_NS = 16
_NW = _NC * _NS
_C = 128
_NCHUNK = _E // _C
_OWN = 632
_OWN_LAST = _N - (_NS - 1) * _OWN

_mesh = plsc.VectorSubcoreMesh(core_axis_name="c", subcore_axis_name="s")


def _zero_fill(zrow_v, n_lane_groups):
    zero16 = jnp.zeros((16,), jnp.float32)
    rows = zrow_v.shape[0]

    def body(i, carry):
        for k in range(n_lane_groups):
            zrow_v[i, pl.ds(k * 16, 16)] = zero16
        return carry

    lax.fori_loop(0, rows, body, 0)


def _copy_owned(s, src_at, dst_at):
    @pl.when(s < _NS - 1)
    def _():
        pltpu.sync_copy(src_at(s * _OWN, _OWN), dst_at(s * _OWN, _OWN))

    @pl.when(s == _NS - 1)
    def _():
        pltpu.sync_copy(src_at(s * _OWN, _OWN_LAST),
                        dst_at(s * _OWN, _OWN_LAST))


def _zero_owned(s, zbuf, sh):
    base = s * _OWN
    for k in range(4):
        pltpu.sync_copy(zbuf, sh.at[pl.ds(base + k * 128, 128)])

    @pl.when(s < _NS - 1)
    def _():
        pltpu.sync_copy(zbuf.at[pl.ds(0, _OWN - 512)],
                        sh.at[pl.ds(base + 512, _OWN - 512)])

    @pl.when(s == _NS - 1)
    def _():
        pltpu.sync_copy(zbuf.at[pl.ds(0, _OWN_LAST - 512)],
                        sh.at[pl.ds(base + 512, _OWN_LAST - 512)])


def _sc_agg_body(h_hbm, src_hbm, dst_hbm, out_hbm, src_v, dst_v, rows_v,
                 agg_sh, sem):
    c = lax.axis_index("c")
    s = lax.axis_index("s")
    wid = s * _NC + c

    _zero_fill(rows_v, _D // 16)
    _zero_owned(s, rows_v, agg_sh)
    plsc.subcore_barrier()

    n_i = (_NCHUNK - wid + _NW - 1) // _NW

    def body(i, carry):
        base = (wid + i * _NW) * _C
        pltpu.sync_copy(src_hbm.at[pl.ds(base, _C)], src_v)
        pltpu.sync_copy(dst_hbm.at[pl.ds(base, _C)], dst_v)
        pltpu.async_copy(h_hbm.at[src_v], rows_v, sem).wait()
        pltpu.sync_copy(rows_v, agg_sh.at[dst_v], add=True)
        return carry

    lax.fori_loop(0, n_i, body, 0)
    plsc.subcore_barrier()

    _copy_owned(s, lambda b, n: agg_sh.at[pl.ds(b, n)],
                lambda b, n: out_hbm.at[pl.ds(c * _N + b, n)])


_sc_agg = pl.kernel(
    _sc_agg_body,
    out_type=jax.ShapeDtypeStruct((_NC * _N, _D), jnp.float32),
    mesh=_mesh,
    scratch_types=[
        pltpu.VMEM((_C,), jnp.int32),
        pltpu.VMEM((_C,), jnp.int32),
        pltpu.VMEM((_C, _D), jnp.float32),
        pltpu.VMEM_SHARED((_N, _D), jnp.float32),
        pltpu.SemaphoreType.DMA,
    ],
)

_CW = 128


def _sc_count_body(dst_hbm, out_hbm, dst_v, ones_v, cnt_sh, sem):
    c = lax.axis_index("c")
    s = lax.axis_index("s")
    wid = s * _NC + c

    _zero_fill(ones_v, _CW // 16)
    _zero_owned(s, ones_v, cnt_sh)

    one16 = jnp.ones((16,), jnp.float32)

    def ones_body(i, carry):
        for k in range(_CW // 16):
            ones_v[i, pl.ds(k * 16, 16)] = one16
        return carry

    lax.fori_loop(0, _C, ones_body, 0)
    plsc.subcore_barrier()

    n_i = (_NCHUNK - wid + _NW - 1) // _NW

    def body(i, carry):
        base = (wid + i * _NW) * _C
        pltpu.sync_copy(dst_hbm.at[pl.ds(base, _C)], dst_v)
        pltpu.sync_copy(ones_v, cnt_sh.at[dst_v], add=True)
        return carry

    lax.fori_loop(0, n_i, body, 0)
    plsc.subcore_barrier()

    _copy_owned(s, lambda b, n: cnt_sh.at[pl.ds(b, n)],
                lambda b, n: out_hbm.at[pl.ds(c * _N + b, n)])


_sc_count = pl.kernel(
    _sc_count_body,
    out_type=jax.ShapeDtypeStruct((_NC * _N, _CW), jnp.float32),
    mesh=_mesh,
    scratch_types=[
        pltpu.VMEM((_C,), jnp.int32),
        pltpu.VMEM((_C, _CW), jnp.float32),
        pltpu.VMEM_SHARED((_N, _CW), jnp.float32),
        pltpu.SemaphoreType.DMA,
    ],
)

_R = 2000


def _tc_layer_body(bn_relu, p_ref, cnt_ref, h_ref, wl_ref, wr_ref, b_ref,
                   g_ref, be_ref, out_ref):
    cnt = cnt_ref[0] + cnt_ref[1]
    rcp = 1.0 / jnp.maximum(cnt, 1.0)
    agg = (p_ref[0] + p_ref[1]) * rcp
    v = jnp.dot(agg, wl_ref[...], preferred_element_type=jnp.float32)
    v = v + jnp.dot(h_ref[...], wr_ref[...], preferred_element_type=jnp.float32)
    v = v + b_ref[...]
    if bn_relu:
        inv = np.float32(1.0) / np.sqrt(np.float32(1.0) + np.float32(_EPS))
        v = v * (g_ref[...] * inv) + be_ref[...]
        v = jnp.maximum(v, 0.0)
    out_ref[...] = v


def _tc_layer(p, cnt_p, h, wl, b, wr, g, be, bn_relu):
    grid = (_N // _R,)
    return pl.pallas_call(
        functools.partial(_tc_layer_body, bn_relu),
        grid=grid,
        in_specs=[
            pl.BlockSpec((_NC, _R, _D), lambda i: (0, i, 0)),
            pl.BlockSpec((_NC, _R, _CW), lambda i: (0, i, 0)),
            pl.BlockSpec((_R, _D), lambda i: (i, 0)),
            pl.BlockSpec((_D, _D), lambda i: (0, 0)),
            pl.BlockSpec((_D, _D), lambda i: (0, 0)),
            pl.BlockSpec((1, _D), lambda i: (0, 0)),
            pl.BlockSpec((1, _D), lambda i: (0, 0)),
            pl.BlockSpec((1, _D), lambda i: (0, 0)),
        ],
        out_specs=pl.BlockSpec((_R, _D), lambda i: (i, 0)),
        out_shape=jax.ShapeDtypeStruct((_N, _D), jnp.float32),
    )(p, cnt_p, h, wl, wr, b.reshape(1, _D), g.reshape(1, _D),
      be.reshape(1, _D))


def kernel(x, edge_index, W1l, b1, W1r, g1, be1, W2l, b2, W2r, g2, be2,
           W3l, b3, W3r):
    src = edge_index[0].astype(jnp.int32)
    dst = edge_index[1].astype(jnp.int32)

    cnt_p = _sc_count(dst).reshape(_NC, _N, _CW)

    p1 = _sc_agg(x, src, dst).reshape(_NC, _N, _D)
    h1 = _tc_layer(p1, cnt_p, x, W1l, b1, W1r, g1, be1, True)

    p2 = _sc_agg(h1, src, dst).reshape(_NC, _N, _D)
    h2 = _tc_layer(p2, cnt_p, h1, W2l, b2, W2r, g2, be2, True)

    p3 = _sc_agg(h2, src, dst).reshape(_NC, _N, _D)
    h3 = _tc_layer(p3, cnt_p, h2, W3l, b3, W3r, g1, be1, False)
    return h3


# speedup vs baseline: 8.5646x; 1.5628x over previous
import functools

import jax
import jax.numpy as jnp
import numpy as np
from jax import lax
from jax.experimental import pallas as pl
from jax.experimental.pallas import tpu as pltpu
from jax.experimental.pallas import tpu_sc as plsc

_N = 10000
_E = 320000
_D = 128
_EPS = 1e-5

_NC = 2
_NS = 16
_NW = _NC * _NS
_C = 128
_NCHUNK = _E // _C
_OWN = 632
_OWN_LAST = _N - (_NS - 1) * _OWN

_mesh = plsc.VectorSubcoreMesh(core_axis_name="c", subcore_axis_name="s")


def _zero_fill(zrow_v, n_lane_groups):
    zero16 = jnp.zeros((16,), jnp.float32)
    rows = zrow_v.shape[0]

    def body(i, carry):
        for k in range(n_lane_groups):
            zrow_v[i, pl.ds(k * 16, 16)] = zero16
        return carry

    lax.fori_loop(0, rows, body, 0)


def _copy_owned(s, src_at, dst_at):
    @pl.when(s < _NS - 1)
    def _():
        pltpu.sync_copy(src_at(s * _OWN, _OWN), dst_at(s * _OWN, _OWN))

    @pl.when(s == _NS - 1)
    def _():
        pltpu.sync_copy(src_at(s * _OWN, _OWN_LAST),
                        dst_at(s * _OWN, _OWN_LAST))


def _zero_owned(s, zbuf, sh):
    base = s * _OWN
    for k in range(4):
        pltpu.sync_copy(zbuf, sh.at[pl.ds(base + k * 128, 128)])

    @pl.when(s < _NS - 1)
    def _():
        pltpu.sync_copy(zbuf.at[pl.ds(0, _OWN - 512)],
                        sh.at[pl.ds(base + 512, _OWN - 512)])

    @pl.when(s == _NS - 1)
    def _():
        pltpu.sync_copy(zbuf.at[pl.ds(0, _OWN_LAST - 512)],
                        sh.at[pl.ds(base + 512, _OWN_LAST - 512)])


def _sc_agg_body(h_hbm, src_hbm, dst_hbm, out_hbm, src_v0, dst_v0, rows_v0,
                 src_v1, dst_v1, rows_v1, agg_sh, sem0, sem1):
    c = lax.axis_index("c")
    s = lax.axis_index("s")
    wid = s * _NC + c

    _zero_fill(rows_v0, _D // 16)
    _zero_owned(s, rows_v0, agg_sh)
    plsc.subcore_barrier()

    n_i = (_NCHUNK - wid + _NW - 1) // _NW
    bufs = ((src_v0, dst_v0, rows_v0, sem0), (src_v1, dst_v1, rows_v1, sem1))

    def load_and_gather(i, buf):
        src_v, dst_v, rows_v, sem = buf
        base = (wid + i * _NW) * _C
        pltpu.sync_copy(src_hbm.at[pl.ds(base, _C)], src_v)
        pltpu.sync_copy(dst_hbm.at[pl.ds(base, _C)], dst_v)
        pltpu.async_copy(h_hbm.at[src_v], rows_v, sem)

    @pl.when(n_i > 0)
    def _():
        load_and_gather(0, bufs[0])

    def body(i, carry):
        for p in (0, 1):

            @pl.when(lax.bitwise_and(i, 1) == p)
            def _():
                cur, nxt = bufs[p], bufs[1 - p]

                @pl.when(i + 1 < n_i)
                def _():
                    load_and_gather(i + 1, nxt)

                src_v, dst_v, rows_v, sem = cur
                pltpu.make_async_copy(h_hbm.at[src_v], rows_v, sem).wait()
                pltpu.sync_copy(rows_v, agg_sh.at[dst_v], add=True)

        return carry

    lax.fori_loop(0, n_i, body, 0)
    plsc.subcore_barrier()

    _copy_owned(s, lambda b, n: agg_sh.at[pl.ds(b, n)],
                lambda b, n: out_hbm.at[pl.ds(c * _N + b, n)])


_sc_agg = pl.kernel(
    _sc_agg_body,
    out_type=jax.ShapeDtypeStruct((_NC * _N, _D), jnp.float32),
    mesh=_mesh,
    scratch_types=[
        pltpu.VMEM((_C,), jnp.int32),
        pltpu.VMEM((_C,), jnp.int32),
        pltpu.VMEM((_C, _D), jnp.float32),
        pltpu.VMEM((_C,), jnp.int32),
        pltpu.VMEM((_C,), jnp.int32),
        pltpu.VMEM((_C, _D), jnp.float32),
        pltpu.VMEM_SHARED((_N, _D), jnp.float32),
        pltpu.SemaphoreType.DMA,
        pltpu.SemaphoreType.DMA,
    ],
)

_CW = 128


def _sc_count_body(dst_hbm, out_hbm, dst_v0, dst_v1, ones_v, cnt_sh,
                   sem0, sem1):
    c = lax.axis_index("c")
    s = lax.axis_index("s")
    wid = s * _NC + c

    _zero_fill(ones_v, _CW // 16)
    _zero_owned(s, ones_v, cnt_sh)

    one16 = jnp.ones((16,), jnp.float32)

    def ones_body(i, carry):
        for k in range(_CW // 16):
            ones_v[i, pl.ds(k * 16, 16)] = one16
        return carry

    lax.fori_loop(0, _C, ones_body, 0)
    plsc.subcore_barrier()

    n_i = (_NCHUNK - wid + _NW - 1) // _NW
    bufs = ((dst_v0, sem0), (dst_v1, sem1))

    @pl.when(n_i > 0)
    def _():
        pltpu.async_copy(dst_hbm.at[pl.ds(wid * _C, _C)], dst_v0, sem0)

    def body(i, carry):
        for p in (0, 1):

            @pl.when(lax.bitwise_and(i, 1) == p)
            def _():
                cur, nxt = bufs[p], bufs[1 - p]

                @pl.when(i + 1 < n_i)
                def _():
                    base = (wid + (i + 1) * _NW) * _C
                    pltpu.async_copy(dst_hbm.at[pl.ds(base, _C)], nxt[0],
                                     nxt[1])

                dst_v, sem = cur
                base = (wid + i * _NW) * _C
                pltpu.make_async_copy(dst_hbm.at[pl.ds(base, _C)], dst_v,
                                      sem).wait()
                pltpu.sync_copy(ones_v, cnt_sh.at[dst_v], add=True)

        return carry

    lax.fori_loop(0, n_i, body, 0)
    plsc.subcore_barrier()

    _copy_owned(s, lambda b, n: cnt_sh.at[pl.ds(b, n)],
                lambda b, n: out_hbm.at[pl.ds(c * _N + b, n)])


_sc_count = pl.kernel(
    _sc_count_body,
    out_type=jax.ShapeDtypeStruct((_NC * _N, _CW), jnp.float32),
    mesh=_mesh,
    scratch_types=[
        pltpu.VMEM((_C,), jnp.int32),
        pltpu.VMEM((_C,), jnp.int32),
        pltpu.VMEM((_C, _CW), jnp.float32),
        pltpu.VMEM_SHARED((_N, _CW), jnp.float32),
        pltpu.SemaphoreType.DMA,
        pltpu.SemaphoreType.DMA,
    ],
)

_R = 2000


def _tc_layer_body(bn_relu, p_ref, cnt_ref, h_ref, wl_ref, wr_ref, b_ref,
                   g_ref, be_ref, out_ref):
    cnt = cnt_ref[0] + cnt_ref[1]
    rcp = 1.0 / jnp.maximum(cnt, 1.0)
    agg = (p_ref[0] + p_ref[1]) * rcp
    v = jnp.dot(agg, wl_ref[...], preferred_element_type=jnp.float32)
    v = v + jnp.dot(h_ref[...], wr_ref[...], preferred_element_type=jnp.float32)
    v = v + b_ref[...]
    if bn_relu:
        inv = np.float32(1.0) / np.sqrt(np.float32(1.0) + np.float32(_EPS))
        v = v * (g_ref[...] * inv) + be_ref[...]
        v = jnp.maximum(v, 0.0)
    out_ref[...] = v


def _tc_layer(p, cnt_p, h, wl, b, wr, g, be, bn_relu):
    grid = (_N // _R,)
    return pl.pallas_call(
        functools.partial(_tc_layer_body, bn_relu),
        grid=grid,
        in_specs=[
            pl.BlockSpec((_NC, _R, _D), lambda i: (0, i, 0)),
            pl.BlockSpec((_NC, _R, _CW), lambda i: (0, i, 0)),
            pl.BlockSpec((_R, _D), lambda i: (i, 0)),
            pl.BlockSpec((_D, _D), lambda i: (0, 0)),
            pl.BlockSpec((_D, _D), lambda i: (0, 0)),
            pl.BlockSpec((1, _D), lambda i: (0, 0)),
            pl.BlockSpec((1, _D), lambda i: (0, 0)),
            pl.BlockSpec((1, _D), lambda i: (0, 0)),
        ],
        out_specs=pl.BlockSpec((_R, _D), lambda i: (i, 0)),
        out_shape=jax.ShapeDtypeStruct((_N, _D), jnp.float32),
    )(p, cnt_p, h, wl, wr, b.reshape(1, _D), g.reshape(1, _D),
      be.reshape(1, _D))


def kernel(x, edge_index, W1l, b1, W1r, g1, be1, W2l, b2, W2r, g2, be2,
           W3l, b3, W3r):
    src = edge_index[0].astype(jnp.int32)
    dst = edge_index[1].astype(jnp.int32)

    cnt_p = _sc_count(dst).reshape(_NC, _N, _CW)

    p1 = _sc_agg(x, src, dst).reshape(_NC, _N, _D)
    h1 = _tc_layer(p1, cnt_p, x, W1l, b1, W1r, g1, be1, True)

    p2 = _sc_agg(h1, src, dst).reshape(_NC, _N, _D)
    h2 = _tc_layer(p2, cnt_p, h1, W2l, b2, W2r, g2, be2, True)

    p3 = _sc_agg(h2, src, dst).reshape(_NC, _N, _D)
    h3 = _tc_layer(p3, cnt_p, h2, W3l, b3, W3r, g1, be1, False)
    return h3
